# Initial kernel scaffold; baseline (speedup 1.0000x reference)
import jax
import jax.numpy as jnp
from jax.experimental import pallas as pl


def kernel(nodes, edges, senders, receivers, frequencies, msg_w1, msg_b1, msg_w2, msg_b2, upd_w1, upd_b1, upd_w2, upd_b2, out_w1, out_b1, out_w2, out_b2):
    raise NotImplementedError("write your pallas kernel here")


# speedup vs baseline: 1.1305x; 1.1305x over previous
import jax
import jax.numpy as jnp
from jax import lax
from jax.experimental import pallas as pl
from jax.experimental.pallas import tpu as pltpu
from jax.experimental.pallas import tpu_sc as plsc

N = 50000
E = 800000
F = 64
H = 64
NB = 8
L = 4

E_PAD = 819200
IDX_ROWS = E_PAD // 128
N_PAD = 51200
TRASH = 50000

NC, NS = 2, 16
ROWS_PER_WORKER = IDX_ROWS // (NC * NS)
GATHER_ITERS = ROWS_PER_WORKER // 8

NCHUNK = 4
CH = N_PAD // NCHUNK
CH_PER_TILE = CH // NS

BE = 3200
BN = 1000

_mesh = plsc.VectorSubcoreMesh(core_axis_name="c", subcore_axis_name="s")


def _gather_body(tab_hbm, sidx_hbm, ridx_hbm, gs_hbm, gr_hbm,
                 idx_v, rows_v, sem):
    wid = lax.axis_index("s") * NC + lax.axis_index("c")
    base_row = wid * ROWS_PER_WORKER

    def step(it, _):
        row0 = base_row + it * 8
        for idx_hbm, out_hbm in ((sidx_hbm, gs_hbm), (ridx_hbm, gr_hbm)):
            pltpu.sync_copy(idx_hbm.at[pl.ds(pl.multiple_of(row0, 8), 8)],
                            idx_v)
            for hb in range(2):
                for j in range(4):
                    pltpu.async_copy(tab_hbm.at[idx_v.at[hb * 4 + j]],
                                     rows_v.at[pl.ds(j * 128, 128)],
                                     sem).wait()
                off = pl.multiple_of(row0 * 128 + hb * 512, 512)
                pltpu.sync_copy(rows_v, out_hbm.at[pl.ds(off, 512)])
        return 0

    lax.fori_loop(0, GATHER_ITERS, step, 0)


_gather = pl.kernel(
    _gather_body,
    out_type=(jax.ShapeDtypeStruct((E_PAD, 2 * H), jnp.float32),
              jax.ShapeDtypeStruct((E_PAD, 2 * H), jnp.float32)),
    mesh=_mesh,
    scratch_types=[
        pltpu.VMEM((8, 128), jnp.int32),
        pltpu.VMEM((512, 2 * H), jnp.float32),
        pltpu.SemaphoreType.DMA,
    ],
)


def _scatter_body(m_hbm, ridx_hbm, bnd_hbm, agg_hbm,
                  acc, idx_v, lidx_v, rows_v, bnd_v):
    c = lax.axis_index("c")
    t = lax.axis_index("s")

    pltpu.sync_copy(bnd_hbm, bnd_v)

    for cc in range(NCHUNK // NC):
        chunk = c * (NCHUNK // NC) + cc
        node_base = chunk * CH

        def zvec(i, _):
            for q in range(8):
                rows_v[i, pl.ds(q * 16, 16)] = jnp.zeros((16,), jnp.float32)
            return 0
        lax.fori_loop(0, 80, zvec, 0)
        def zcopy(k, _):
            pltpu.sync_copy(
                rows_v.at[pl.ds(0, 80)],
                acc.at[pl.ds(pl.multiple_of(t * CH_PER_TILE + k * 80, 80),
                             80)])
            return 0
        lax.fori_loop(0, CH_PER_TILE // 80, zcopy, 0)

        @pl.when(t == 0)
        def _():
            pltpu.sync_copy(rows_v.at[pl.ds(0, 8)], acc.at[pl.ds(CH, 8)])
        plsc.subcore_barrier()

        lo = jnp.where(c == 0, bnd_v[2 * cc, pl.ds(0, 16)][0],
                       bnd_v[2 * (cc + NCHUNK // NC), pl.ds(0, 16)][0])
        hi = jnp.where(c == 0, bnd_v[2 * cc + 1, pl.ds(0, 16)][0],
                       bnd_v[2 * (cc + NCHUNK // NC) + 1, pl.ds(0, 16)][0])
        nit = (hi - lo + NS - 1) // NS

        def step(i, _):
            blk = lo + t + i * NS

            @pl.when(blk < hi)
            def _():
                row0 = blk * 8
                pltpu.sync_copy(
                    ridx_hbm.at[pl.ds(pl.multiple_of(row0, 8), 8)], idx_v)
                for j in range(8):
                    for q in range(8):
                        v = idx_v[j, pl.ds(q * 16, 16)] - node_base
                        oob = (v < 0) | (v >= CH)
                        lidx_v[j, pl.ds(q * 16, 16)] = jnp.where(oob, CH, v)
                for j in range(8):
                    off = pl.multiple_of(row0 * 128 + j * 128, 128)
                    pltpu.sync_copy(m_hbm.at[pl.ds(off, 128)], rows_v)
                    pltpu.sync_copy(rows_v, acc.at[lidx_v.at[j]], add=True)
            return 0

        lax.fori_loop(0, nit, step, 0)
        plsc.subcore_barrier()

        def wb(k, _):
            r0 = pl.multiple_of(t * CH_PER_TILE + k * 160, 160)
            pltpu.sync_copy(acc.at[pl.ds(r0, 160)],
                            agg_hbm.at[pl.ds(node_base + r0, 160)])
            return 0
        lax.fori_loop(0, CH_PER_TILE // 160, wb, 0)
        plsc.subcore_barrier()


_scatter = pl.kernel(
    _scatter_body,
    out_type=jax.ShapeDtypeStruct((N_PAD, 2 * H), jnp.float32),
    mesh=_mesh,
    scratch_types=[
        pltpu.VMEM_SHARED((CH + 8, 2 * H), jnp.float32),
        pltpu.VMEM((8, 128), jnp.int32),
        pltpu.VMEM((8, 128), jnp.int32),
        pltpu.VMEM((128, 2 * H), jnp.float32),
        pltpu.VMEM((8, 16), jnp.int32),
    ],
)


def _swish(x):
    return x * jax.nn.sigmoid(x)


def _bessel_kernel(edges_ref, freq_ref, out_ref):
    e = edges_ref[...]
    d = jnp.sqrt(e[:, 0:1] ** 2 + e[:, 1:2] ** 2 + e[:, 2:3] ** 2)
    scaled = jnp.clip(d * 0.25, 0.0, 1.0)
    x = scaled * freq_ref[...]
    out_ref[...] = jnp.sin(x) / x


def _bessel_call(edges_pad, freq):
    return pl.pallas_call(
        _bessel_kernel,
        grid=(E_PAD // BE,),
        in_specs=[pl.BlockSpec((BE, 3), lambda i: (i, 0)),
                  pl.BlockSpec((1, NB), lambda i: (0, 0))],
        out_specs=pl.BlockSpec((BE, NB), lambda i: (i, 0)),
        out_shape=jax.ShapeDtypeStruct((E_PAD, NB), jnp.float32),
    )(edges_pad, freq)


def _proj_kernel(h_ref, ws_ref, wr_ref, t_ref):
    h = h_ref[...]
    t_ref[...] = jnp.concatenate(
        [jnp.dot(h, ws_ref[...], preferred_element_type=jnp.float32, precision=lax.Precision.HIGHEST),
         jnp.dot(h, wr_ref[...], preferred_element_type=jnp.float32, precision=lax.Precision.HIGHEST)],
        axis=1)


def _proj_call(h, ws, wr):
    return pl.pallas_call(
        _proj_kernel,
        grid=(N // BN,),
        in_specs=[pl.BlockSpec((BN, F), lambda i: (i, 0)),
                  pl.BlockSpec((F, H), lambda i: (0, 0)),
                  pl.BlockSpec((F, H), lambda i: (0, 0))],
        out_specs=pl.BlockSpec((BN, 2 * H), lambda i: (i, 0)),
        out_shape=jax.ShapeDtypeStruct((N, 2 * H), jnp.float32),
    )(h, ws, wr)


def _edge_kernel(gs_ref, gr_ref, rad_ref, wc_ref, b1_ref, w2_ref, b2_ref,
                 m_ref):
    pre = (gs_ref[:, :H] + gr_ref[:, H:]
           + jnp.dot(rad_ref[...], wc_ref[...],
                     preferred_element_type=jnp.float32, precision=lax.Precision.HIGHEST) + b1_ref[...])
    m = jnp.dot(_swish(pre), w2_ref[...],
                preferred_element_type=jnp.float32, precision=lax.Precision.HIGHEST) + b2_ref[...]
    m_ref[...] = jnp.concatenate([m, jnp.zeros((BE, H), jnp.float32)], axis=1)


def _edge_call(gs_full, gr_full, rad, wc, b1, w2, b2):
    return pl.pallas_call(
        _edge_kernel,
        grid=(E_PAD // BE,),
        in_specs=[pl.BlockSpec((BE, 2 * H), lambda i: (i, 0)),
                  pl.BlockSpec((BE, 2 * H), lambda i: (i, 0)),
                  pl.BlockSpec((BE, NB), lambda i: (i, 0)),
                  pl.BlockSpec((NB, H), lambda i: (0, 0)),
                  pl.BlockSpec((1, H), lambda i: (0, 0)),
                  pl.BlockSpec((H, H), lambda i: (0, 0)),
                  pl.BlockSpec((1, H), lambda i: (0, 0))],
        out_specs=pl.BlockSpec((BE, 2 * H), lambda i: (i, 0)),
        out_shape=jax.ShapeDtypeStruct((E_PAD, 2 * H), jnp.float32),
    )(gs_full, gr_full, rad, wc, b1, w2, b2)


def _node_kernel(h_ref, a_ref, uh_ref, ua_ref, b1_ref, u2_ref, b2_ref,
                 out_ref):
    pre = (jnp.dot(h_ref[...], uh_ref[...],
                   preferred_element_type=jnp.float32, precision=lax.Precision.HIGHEST)
           + jnp.dot(a_ref[:, :H], ua_ref[...],
                     preferred_element_type=jnp.float32, precision=lax.Precision.HIGHEST)
           + b1_ref[...])
    out_ref[...] = jnp.dot(_swish(pre), u2_ref[...],
                           preferred_element_type=jnp.float32, precision=lax.Precision.HIGHEST) + b2_ref[...]


def _node_call(h, agg, uh, ua, b1, u2, b2):
    return pl.pallas_call(
        _node_kernel,
        grid=(N // BN,),
        in_specs=[pl.BlockSpec((BN, F), lambda i: (i, 0)),
                  pl.BlockSpec((BN, 2 * H), lambda i: (i, 0)),
                  pl.BlockSpec((F, H), lambda i: (0, 0)),
                  pl.BlockSpec((H, H), lambda i: (0, 0)),
                  pl.BlockSpec((1, H), lambda i: (0, 0)),
                  pl.BlockSpec((H, H), lambda i: (0, 0)),
                  pl.BlockSpec((1, H), lambda i: (0, 0))],
        out_specs=pl.BlockSpec((BN, H), lambda i: (i, 0)),
        out_shape=jax.ShapeDtypeStruct((N, H), jnp.float32),
    )(h, agg, uh, ua, b1, u2, b2)


def _readout_kernel(h_ref, w1_ref, b1_ref, w2_ref, b2_ref, out_ref, acc_ref):
    i = pl.program_id(0)

    @pl.when(i == 0)
    def _():
        acc_ref[...] = jnp.zeros_like(acc_ref)

    sw = _swish(jnp.dot(h_ref[...], w1_ref[...],
                        preferred_element_type=jnp.float32, precision=lax.Precision.HIGHEST) + b1_ref[...])
    acc_ref[...] += jnp.sum(sw, axis=0, keepdims=True)

    @pl.when(i == pl.num_programs(0) - 1)
    def _():
        out_ref[...] = (jnp.dot(acc_ref[...], w2_ref[...],
                                preferred_element_type=jnp.float32, precision=lax.Precision.HIGHEST)
                        + float(N) * b2_ref[...])


def _readout_call(h, w1, b1, w2, b2):
    return pl.pallas_call(
        _readout_kernel,
        grid=(N // BN,),
        in_specs=[pl.BlockSpec((BN, H), lambda i: (i, 0)),
                  pl.BlockSpec((H, H), lambda i: (0, 0)),
                  pl.BlockSpec((1, H), lambda i: (0, 0)),
                  pl.BlockSpec((H, 1), lambda i: (0, 0)),
                  pl.BlockSpec((1, 1), lambda i: (0, 0))],
        out_specs=pl.BlockSpec((1, 1), lambda i: (0, 0)),
        out_shape=jax.ShapeDtypeStruct((1, 1), jnp.float32),
        scratch_shapes=[pltpu.VMEM((1, H), jnp.float32)],
    )(h, w1, b1, w2, b2)


def kernel(nodes, edges, senders, receivers, frequencies,
           msg_w1, msg_b1, msg_w2, msg_b2,
           upd_w1, upd_b1, upd_w2, upd_b2,
           out_w1, out_b1, out_w2, out_b2):
    npad = E_PAD - E
    sidx = jnp.concatenate(
        [senders, jnp.zeros((npad,), jnp.int32)]).reshape(IDX_ROWS, 128)
    ridx = jnp.concatenate(
        [receivers, jnp.full((npad,), TRASH, jnp.int32)]).reshape(IDX_ROWS, 128)
    edges_pad = jnp.concatenate(
        [edges, jnp.tile(jnp.array([[1.0, 0.0, 0.0]], jnp.float32),
                         (npad, 1))])
    freq = frequencies.reshape(1, NB)

    eb = jnp.searchsorted(
        receivers, jnp.arange(1, NCHUNK, dtype=jnp.int32) * CH).astype(
            jnp.int32)
    starts = jnp.concatenate([jnp.zeros((1,), jnp.int32), eb])
    ends = jnp.concatenate([eb, jnp.full((1,), E_PAD, jnp.int32)])
    los = starts // 1024
    his = (ends + 1023) // 1024
    bnd = jnp.stack([los, his], axis=1).reshape(2 * NCHUNK, 1)
    bnd = jnp.broadcast_to(bnd, (2 * NCHUNK, 16))

    rad = _bessel_call(edges_pad, freq)

    h = nodes
    for i in range(L):
        w1 = msg_w1[i]
        tab = _proj_call(h, w1[:F], w1[F:2 * F])
        gs_full, gr_full = _gather(tab, sidx, ridx)
        m = _edge_call(gs_full, gr_full, rad, w1[2 * F:],
                       msg_b1[i].reshape(1, H), msg_w2[i],
                       msg_b2[i].reshape(1, H))
        agg = _scatter(m, ridx, bnd)
        h = _node_call(h, agg, upd_w1[i][:F], upd_w1[i][F:],
                       upd_b1[i].reshape(1, H), upd_w2[i],
                       upd_b2[i].reshape(1, H))

    out = _readout_call(h, out_w1, out_b1.reshape(1, H), out_w2,
                        out_b2.reshape(1, 1))
    return out.reshape(1)


# speedup vs baseline: 1.2231x; 1.0819x over previous
import jax
import jax.numpy as jnp
from jax import lax
from jax.experimental import pallas as pl
from jax.experimental.pallas import tpu as pltpu
from jax.experimental.pallas import tpu_sc as plsc

N = 50000
E = 800000
F = 64
H = 64
NB = 8
L = 4

E_PAD = 819200
IDX_ROWS = E_PAD // 128
N_PAD = 51200
TRASH = 50000

NC, NS = 2, 16
ROWS_PER_WORKER = IDX_ROWS // (NC * NS)
GATHER_ITERS = ROWS_PER_WORKER // 8

NCHUNK = 8
CH = N_PAD // NCHUNK
CH_PER_TILE = CH // NS

BE = 3200
BN = 1000

_mesh = plsc.VectorSubcoreMesh(core_axis_name="c", subcore_axis_name="s")


def _gather_body(tab_hbm, sidx_hbm, ridx_hbm, gs_hbm, gr_hbm,
                 idx_v, rows_a, rows_b, sem_g, sem_wa, sem_wb):
    wid = lax.axis_index("s") * NC + lax.axis_index("c")
    base_row = wid * ROWS_PER_WORKER
    bufs = (rows_a, rows_b)
    wsems = (sem_wa, sem_wb)

    def step(it, _):
        row0 = base_row + it * 8
        for idx_hbm, out_hbm in ((sidx_hbm, gs_hbm), (ridx_hbm, gr_hbm)):
            pltpu.sync_copy(idx_hbm.at[pl.ds(pl.multiple_of(row0, 8), 8)],
                            idx_v)
            writes = []
            for g in range(4):
                buf = bufs[g % 2]
                if len(writes) >= 2:
                    writes[g - 2].wait()
                gathers = [
                    pltpu.async_copy(tab_hbm.at[idx_v.at[2 * g + j]],
                                     buf.at[pl.ds(j * 128, 128)], sem_g)
                    for j in range(2)]
                for cp in gathers:
                    cp.wait()
                off = pl.multiple_of(row0 * 128 + g * 256, 256)
                writes.append(
                    pltpu.async_copy(buf, out_hbm.at[pl.ds(off, 256)],
                                     wsems[g % 2]))
            writes[2].wait()
            writes[3].wait()
        return 0

    lax.fori_loop(0, GATHER_ITERS, step, 0)


_gather = pl.kernel(
    _gather_body,
    out_type=(jax.ShapeDtypeStruct((E_PAD, 2 * H), jnp.float32),
              jax.ShapeDtypeStruct((E_PAD, 2 * H), jnp.float32)),
    mesh=_mesh,
    scratch_types=[
        pltpu.VMEM((8, 128), jnp.int32),
        pltpu.VMEM((256, 2 * H), jnp.float32),
        pltpu.VMEM((256, 2 * H), jnp.float32),
        pltpu.SemaphoreType.DMA,
        pltpu.SemaphoreType.DMA,
        pltpu.SemaphoreType.DMA,
    ],
)


def _scatter_body(m_hbm, ridx_hbm, bnd_hbm, agg_hbm,
                  acc, idx_v, lidx_v, rows_a, rows_b, bnd_v,
                  sem_a, sem_b):
    c = lax.axis_index("c")
    t = lax.axis_index("s")
    bufs = (rows_a, rows_b)
    sems = (sem_a, sem_b)

    pltpu.sync_copy(bnd_hbm, bnd_v)

    for cc in range(NCHUNK // NC):
        chunk = c * (NCHUNK // NC) + cc
        node_base = chunk * CH

        def zvec(i, _):
            for q in range(8):
                rows_a[i, pl.ds(q * 16, 16)] = jnp.zeros((16,), jnp.float32)
            return 0
        lax.fori_loop(0, 80, zvec, 0)
        def zcopy(k, _):
            pltpu.sync_copy(
                rows_a.at[pl.ds(0, 80)],
                acc.at[pl.ds(pl.multiple_of(t * CH_PER_TILE + k * 80, 80),
                             80)])
            return 0
        lax.fori_loop(0, CH_PER_TILE // 80, zcopy, 0)

        @pl.when(t == 0)
        def _():
            pltpu.sync_copy(rows_a.at[pl.ds(0, 8)], acc.at[pl.ds(CH, 8)])
        plsc.subcore_barrier()

        lo = jnp.where(c == 0, bnd_v[2 * cc, pl.ds(0, 16)][0],
                       bnd_v[2 * (cc + NCHUNK // NC), pl.ds(0, 16)][0])
        hi = jnp.where(c == 0, bnd_v[2 * cc + 1, pl.ds(0, 16)][0],
                       bnd_v[2 * (cc + NCHUNK // NC) + 1, pl.ds(0, 16)][0])
        nit = (hi - lo + NS - 1) // NS

        def step(i, _):
            blk = lo + t + i * NS

            @pl.when(blk < hi)
            def _():
                row0 = blk * 8
                pltpu.sync_copy(
                    ridx_hbm.at[pl.ds(pl.multiple_of(row0, 8), 8)], idx_v)
                for j in range(8):
                    for q in range(8):
                        v = idx_v[j, pl.ds(q * 16, 16)] - node_base
                        oob = (v < 0) | (v >= CH)
                        lidx_v[j, pl.ds(q * 16, 16)] = jnp.where(oob, CH, v)
                cps = [pltpu.async_copy(
                    m_hbm.at[pl.ds(pl.multiple_of(row0 * 128, 128), 128)],
                    rows_a, sem_a)]
                for j in range(8):
                    if j < 7:
                        off = pl.multiple_of(row0 * 128 + (j + 1) * 128, 128)
                        cps.append(pltpu.async_copy(
                            m_hbm.at[pl.ds(off, 128)],
                            bufs[(j + 1) % 2], sems[(j + 1) % 2]))
                    cps[j].wait()
                    pltpu.sync_copy(bufs[j % 2], acc.at[lidx_v.at[j]],
                                    add=True)
            return 0

        lax.fori_loop(0, nit, step, 0)
        plsc.subcore_barrier()

        def wb(k, _):
            r0 = pl.multiple_of(t * CH_PER_TILE + k * 80, 80)
            pltpu.sync_copy(acc.at[pl.ds(r0, 80)],
                            agg_hbm.at[pl.ds(node_base + r0, 80)])
            return 0
        lax.fori_loop(0, CH_PER_TILE // 80, wb, 0)
        plsc.subcore_barrier()


_scatter = pl.kernel(
    _scatter_body,
    out_type=jax.ShapeDtypeStruct((N_PAD, 2 * H), jnp.float32),
    mesh=_mesh,
    scratch_types=[
        pltpu.VMEM_SHARED((CH + 8, 2 * H), jnp.float32),
        pltpu.VMEM((8, 128), jnp.int32),
        pltpu.VMEM((8, 128), jnp.int32),
        pltpu.VMEM((128, 2 * H), jnp.float32),
        pltpu.VMEM((128, 2 * H), jnp.float32),
        pltpu.VMEM((2 * NCHUNK, 16), jnp.int32),
        pltpu.SemaphoreType.DMA,
        pltpu.SemaphoreType.DMA,
    ],
)


def _swish(x):
    return x * jax.nn.sigmoid(x)


def _bessel_kernel(edges_ref, freq_ref, out_ref):
    e = edges_ref[...]
    d = jnp.sqrt(e[:, 0:1] ** 2 + e[:, 1:2] ** 2 + e[:, 2:3] ** 2)
    scaled = jnp.clip(d * 0.25, 0.0, 1.0)
    x = scaled * freq_ref[...]
    out_ref[...] = jnp.sin(x) / x


def _bessel_call(edges_pad, freq):
    return pl.pallas_call(
        _bessel_kernel,
        grid=(E_PAD // BE,),
        in_specs=[pl.BlockSpec((BE, 3), lambda i: (i, 0)),
                  pl.BlockSpec((1, NB), lambda i: (0, 0))],
        out_specs=pl.BlockSpec((BE, NB), lambda i: (i, 0)),
        out_shape=jax.ShapeDtypeStruct((E_PAD, NB), jnp.float32),
    )(edges_pad, freq)


def _proj_kernel(h_ref, ws_ref, wr_ref, t_ref):
    h = h_ref[...]
    t_ref[...] = jnp.concatenate(
        [jnp.dot(h, ws_ref[...], preferred_element_type=jnp.float32, precision=lax.Precision.HIGHEST),
         jnp.dot(h, wr_ref[...], preferred_element_type=jnp.float32, precision=lax.Precision.HIGHEST)],
        axis=1)


def _proj_call(h, ws, wr):
    return pl.pallas_call(
        _proj_kernel,
        grid=(N // BN,),
        in_specs=[pl.BlockSpec((BN, F), lambda i: (i, 0)),
                  pl.BlockSpec((F, H), lambda i: (0, 0)),
                  pl.BlockSpec((F, H), lambda i: (0, 0))],
        out_specs=pl.BlockSpec((BN, 2 * H), lambda i: (i, 0)),
        out_shape=jax.ShapeDtypeStruct((N, 2 * H), jnp.float32),
    )(h, ws, wr)


def _edge_kernel(gs_ref, gr_ref, rad_ref, wc_ref, b1_ref, w2_ref, b2_ref,
                 m_ref):
    pre = (gs_ref[:, :H] + gr_ref[:, H:]
           + jnp.dot(rad_ref[...], wc_ref[...],
                     preferred_element_type=jnp.float32, precision=lax.Precision.HIGHEST) + b1_ref[...])
    m = jnp.dot(_swish(pre), w2_ref[...],
                preferred_element_type=jnp.float32, precision=lax.Precision.HIGHEST) + b2_ref[...]
    m_ref[...] = jnp.concatenate([m, jnp.zeros((BE, H), jnp.float32)], axis=1)


def _edge_call(gs_full, gr_full, rad, wc, b1, w2, b2):
    return pl.pallas_call(
        _edge_kernel,
        grid=(E_PAD // BE,),
        in_specs=[pl.BlockSpec((BE, 2 * H), lambda i: (i, 0)),
                  pl.BlockSpec((BE, 2 * H), lambda i: (i, 0)),
                  pl.BlockSpec((BE, NB), lambda i: (i, 0)),
                  pl.BlockSpec((NB, H), lambda i: (0, 0)),
                  pl.BlockSpec((1, H), lambda i: (0, 0)),
                  pl.BlockSpec((H, H), lambda i: (0, 0)),
                  pl.BlockSpec((1, H), lambda i: (0, 0))],
        out_specs=pl.BlockSpec((BE, 2 * H), lambda i: (i, 0)),
        out_shape=jax.ShapeDtypeStruct((E_PAD, 2 * H), jnp.float32),
    )(gs_full, gr_full, rad, wc, b1, w2, b2)


def _node_kernel(h_ref, a_ref, uh_ref, ua_ref, b1_ref, u2_ref, b2_ref,
                 out_ref):
    pre = (jnp.dot(h_ref[...], uh_ref[...],
                   preferred_element_type=jnp.float32, precision=lax.Precision.HIGHEST)
           + jnp.dot(a_ref[:, :H], ua_ref[...],
                     preferred_element_type=jnp.float32, precision=lax.Precision.HIGHEST)
           + b1_ref[...])
    out_ref[...] = jnp.dot(_swish(pre), u2_ref[...],
                           preferred_element_type=jnp.float32, precision=lax.Precision.HIGHEST) + b2_ref[...]


def _node_call(h, agg, uh, ua, b1, u2, b2):
    return pl.pallas_call(
        _node_kernel,
        grid=(N // BN,),
        in_specs=[pl.BlockSpec((BN, F), lambda i: (i, 0)),
                  pl.BlockSpec((BN, 2 * H), lambda i: (i, 0)),
                  pl.BlockSpec((F, H), lambda i: (0, 0)),
                  pl.BlockSpec((H, H), lambda i: (0, 0)),
                  pl.BlockSpec((1, H), lambda i: (0, 0)),
                  pl.BlockSpec((H, H), lambda i: (0, 0)),
                  pl.BlockSpec((1, H), lambda i: (0, 0))],
        out_specs=pl.BlockSpec((BN, H), lambda i: (i, 0)),
        out_shape=jax.ShapeDtypeStruct((N, H), jnp.float32),
    )(h, agg, uh, ua, b1, u2, b2)


def _readout_kernel(h_ref, w1_ref, b1_ref, w2_ref, b2_ref, out_ref, acc_ref):
    i = pl.program_id(0)

    @pl.when(i == 0)
    def _():
        acc_ref[...] = jnp.zeros_like(acc_ref)

    sw = _swish(jnp.dot(h_ref[...], w1_ref[...],
                        preferred_element_type=jnp.float32, precision=lax.Precision.HIGHEST) + b1_ref[...])
    acc_ref[...] += jnp.sum(sw, axis=0, keepdims=True)

    @pl.when(i == pl.num_programs(0) - 1)
    def _():
        out_ref[...] = (jnp.dot(acc_ref[...], w2_ref[...],
                                preferred_element_type=jnp.float32, precision=lax.Precision.HIGHEST)
                        + float(N) * b2_ref[...])


def _readout_call(h, w1, b1, w2, b2):
    return pl.pallas_call(
        _readout_kernel,
        grid=(N // BN,),
        in_specs=[pl.BlockSpec((BN, H), lambda i: (i, 0)),
                  pl.BlockSpec((H, H), lambda i: (0, 0)),
                  pl.BlockSpec((1, H), lambda i: (0, 0)),
                  pl.BlockSpec((H, 1), lambda i: (0, 0)),
                  pl.BlockSpec((1, 1), lambda i: (0, 0))],
        out_specs=pl.BlockSpec((1, 1), lambda i: (0, 0)),
        out_shape=jax.ShapeDtypeStruct((1, 1), jnp.float32),
        scratch_shapes=[pltpu.VMEM((1, H), jnp.float32)],
    )(h, w1, b1, w2, b2)


def kernel(nodes, edges, senders, receivers, frequencies,
           msg_w1, msg_b1, msg_w2, msg_b2,
           upd_w1, upd_b1, upd_w2, upd_b2,
           out_w1, out_b1, out_w2, out_b2):
    npad = E_PAD - E
    sidx = jnp.concatenate(
        [senders, jnp.zeros((npad,), jnp.int32)]).reshape(IDX_ROWS, 128)
    ridx = jnp.concatenate(
        [receivers, jnp.full((npad,), TRASH, jnp.int32)]).reshape(IDX_ROWS, 128)
    edges_pad = jnp.concatenate(
        [edges, jnp.tile(jnp.array([[1.0, 0.0, 0.0]], jnp.float32),
                         (npad, 1))])
    freq = frequencies.reshape(1, NB)

    eb = jnp.searchsorted(
        receivers, jnp.arange(1, NCHUNK, dtype=jnp.int32) * CH).astype(
            jnp.int32)
    starts = jnp.concatenate([jnp.zeros((1,), jnp.int32), eb])
    ends = jnp.concatenate([eb, jnp.full((1,), E_PAD, jnp.int32)])
    los = starts // 1024
    his = (ends + 1023) // 1024
    bnd = jnp.stack([los, his], axis=1).reshape(2 * NCHUNK, 1)
    bnd = jnp.broadcast_to(bnd, (2 * NCHUNK, 16))

    rad = _bessel_call(edges_pad, freq)

    h = nodes
    for i in range(L):
        w1 = msg_w1[i]
        tab = _proj_call(h, w1[:F], w1[F:2 * F])
        gs_full, gr_full = _gather(tab, sidx, ridx)
        m = _edge_call(gs_full, gr_full, rad, w1[2 * F:],
                       msg_b1[i].reshape(1, H), msg_w2[i],
                       msg_b2[i].reshape(1, H))
        agg = _scatter(m, ridx, bnd)
        h = _node_call(h, agg, upd_w1[i][:F], upd_w1[i][F:],
                       upd_b1[i].reshape(1, H), upd_w2[i],
                       upd_b2[i].reshape(1, H))

    out = _readout_call(h, out_w1, out_b1.reshape(1, H), out_w2,
                        out_b2.reshape(1, 1))
    return out.reshape(1)
